# baseline (speedup 1.0000x reference)
import functools

import jax
import jax.numpy as jnp
from jax import lax
from jax.experimental import pallas as pl
from jax.experimental.pallas import tpu as pltpu
from jax.experimental.pallas import tpu_sc as plsc

_B, _F, _T = 64, 128, 4096
_HS = 64
_NC, _NS = 2, 16
_NW = _NC * _NS
_BPW = _B // _NW
_RC = 8
_NCH = _F // _RC
_ML = 16


def _patch_rows(buf, chunk, f, t):
    r0 = chunk * _RC
    zv = jnp.zeros((16,), jnp.float32)
    li = lax.broadcasted_iota(jnp.int32, (16,), 0)
    lo = jnp.maximum(f - r0, 0)
    hi = jnp.minimum(f + _HS - r0, _RC)

    def row_body(r, carry):
        rows = jnp.full((16,), r, jnp.int32)
        for j in range(_HS // 16):
            plsc.store_scatter(buf, [rows, t + j * 16 + li], zv)
        return carry

    lax.fori_loop(lo, hi, row_body, 0)


def _sc_body(x_hbm, meta_hbm, out_hbm, buf0, buf1, meta_v,
             isem0, isem1, osem0, osem1):
    wid = lax.axis_index("s") * _NC + lax.axis_index("c")
    pltpu.sync_copy(meta_hbm, meta_v)
    bufs = (buf0, buf1)
    isems = (isem0, isem1)
    osems = (osem0, osem1)

    for bi in range(_BPW):
        b = wid * _BPW + bi
        mv = meta_v[b]
        f0a, f0b = mv[0], mv[1]
        t0a, t0b = mv[2], mv[3]

        def load(c):
            k = c % 2
            pltpu.make_async_copy(
                x_hbm.at[b, pl.ds(c * _RC, _RC)], bufs[k], isems[k]).start()

        def load_wait(c):
            k = c % 2
            pltpu.make_async_copy(
                x_hbm.at[b, pl.ds(c * _RC, _RC)], bufs[k], isems[k]).wait()

        def store(c):
            k = c % 2
            pltpu.make_async_copy(
                bufs[k], out_hbm.at[b, pl.ds(c * _RC, _RC)], osems[k]).start()

        def store_wait(c):
            k = c % 2
            pltpu.make_async_copy(
                bufs[k], out_hbm.at[b, pl.ds(c * _RC, _RC)], osems[k]).wait()

        load(0)
        for c in range(_NCH):
            if c + 1 < _NCH:
                if c - 1 >= 0:
                    store_wait(c - 1)
                load(c + 1)
            load_wait(c)
            _patch_rows(bufs[c % 2], c, f0a, t0a)
            _patch_rows(bufs[c % 2], c, f0b, t0b)
            store(c)
        store_wait(_NCH - 2)
        store_wait(_NCH - 1)


def kernel(x, f0, t0):
    meta = jnp.concatenate([
        f0.astype(jnp.int32),
        t0.astype(jnp.int32),
        jnp.zeros((_B, _ML - 4), jnp.int32),
    ], axis=1)
    mesh = plsc.VectorSubcoreMesh(core_axis_name="c", subcore_axis_name="s")
    fn = functools.partial(
        pl.kernel,
        out_type=jax.ShapeDtypeStruct((_B, _F, _T), jnp.float32),
        mesh=mesh,
        compiler_params=pltpu.CompilerParams(needs_layout_passes=False),
        scratch_types=[
            pltpu.VMEM((_RC, _T), jnp.float32),
            pltpu.VMEM((_RC, _T), jnp.float32),
            pltpu.VMEM((_B, _ML), jnp.int32),
            pltpu.SemaphoreType.DMA,
            pltpu.SemaphoreType.DMA,
            pltpu.SemaphoreType.DMA,
            pltpu.SemaphoreType.DMA,
        ],
    )(_sc_body)
    return fn(x, meta)
